# baseline (speedup 1.0000x reference)
import functools

import jax
import jax.numpy as jnp
from jax import lax
from jax.experimental import pallas as pl
from jax.experimental.pallas import tpu as pltpu
from jax.experimental.pallas import tpu_sc as plsc

NUM_NODES = 100000
NODE_DIM = 128
LAYER_DIM = 64
NUM_LAYER = 8
BATCH = 4096

NC = 2
NS = 16
NW = NC * NS

GB = 2 * BATCH
N_PER_W = GB // NW
CHUNK = 128
N_CHUNKS = N_PER_W // CHUNK


SUB = 64
N_SUB = CHUNK // SUB


def _sc_gather_body(i_hbm, j_hbm, nemb_hbm, out_g_hbm, idx_v, rows_v,
                    isem, gsem, wsem):
  wid = lax.axis_index("s") * NC + lax.axis_index("c")
  base = wid * CHUNK
  st0 = pltpu.async_copy(i_hbm.at[pl.ds(base, CHUNK)], idx_v.at[0], isem)
  st1 = pltpu.async_copy(j_hbm.at[pl.ds(base, CHUNK)], idx_v.at[1], isem)
  st0.wait()
  st1.wait()
  gathers = []
  for h in range(N_CHUNKS):
    for c in range(N_SUB):
      gathers.append(pltpu.async_copy(
          nemb_hbm.at[idx_v.at[h, pl.ds(c * SUB, SUB)]],
          rows_v.at[pl.ds(h * CHUNK + c * SUB, SUB)], gsem))
  writes = []
  for h in range(N_CHUNKS):
    for c in range(N_SUB):
      gathers[h * N_SUB + c].wait()
      writes.append(pltpu.async_copy(
          rows_v.at[pl.ds(h * CHUNK + c * SUB, SUB)],
          out_g_hbm.at[pl.ds(h * BATCH + base + c * SUB, SUB)], wsem))
  for wcp in writes:
    wcp.wait()


@functools.cache
def _sc_gather():
  return pl.kernel(
      _sc_gather_body,
      out_type=jax.ShapeDtypeStruct((GB, NODE_DIM), jnp.float32),
      mesh=plsc.VectorSubcoreMesh(
          core_axis_name="c", subcore_axis_name="s",
          num_cores=NC, num_subcores=NS),
      scratch_types=[
          pltpu.VMEM((N_CHUNKS, CHUNK), jnp.int32),
          pltpu.VMEM((N_PER_W, NODE_DIM), jnp.float32),
          pltpu.SemaphoreType.DMA,
          pltpu.SemaphoreType.DMA,
          pltpu.SemaphoreType.DMA,
      ],
  )


TC_STEPS = 2
TCB = BATCH // TC_STEPS


def _tc_body(gi_ref, gj_ref, l_ref, label_ref, lemb_ref, w_ref, out_ref,
             acc_ref):
  t = pl.program_id(0)
  w = w_ref[...]
  gi = gi_ref[...]
  gj = gj_ref[...]
  lemb = lemb_ref[...]
  pu = jnp.dot(gi + gj, w, preferred_element_type=jnp.float32)
  li = l_ref[...]
  oh = (lax.broadcasted_iota(jnp.int32, (TCB, NUM_LAYER), 1)
        == li).astype(jnp.float32)
  m8 = lax.dot_general(oh, pu, (((0,), (0,)), ((), ())),
                       preferred_element_type=jnp.float32)
  counts = jnp.sum(oh, axis=0, keepdims=True)
  l2 = jnp.sum(lemb * lemb, axis=1, keepdims=True)
  cross = lax.dot_general(gi, gj, (((0,), (0,)), ((), ())),
                          preferred_element_type=jnp.float32)
  gw = jnp.dot(w, w.T, preferred_element_type=jnp.float32)
  part = (jnp.sum(counts * l2.T) + jnp.sum(lemb * m8)
          + jnp.sum(gw * cross))

  @pl.when(t == 0)
  def _init():
    acc_ref[0] = part

  @pl.when(jnp.logical_and(t > 0, t < TC_STEPS - 1))
  def _acc():
    acc_ref[0] += part

  @pl.when(t == TC_STEPS - 1)
  def _fini():
    z = label_ref[...] * (acc_ref[0] + part)
    ls = jnp.minimum(z, 0.0) - jnp.log1p(jnp.exp(-jnp.abs(z)))
    out_ref[...] = (-jnp.sum(ls)).reshape(1, 1)


def kernel(i, j, l, label, n_emb, l_emb, w):
  g = _sc_gather()(i.astype(jnp.int32), j.astype(jnp.int32), n_emb)
  out = pl.pallas_call(
      _tc_body,
      grid=(TC_STEPS,),
      in_specs=[
          pl.BlockSpec((TCB, NODE_DIM), lambda t: (t, 0)),
          pl.BlockSpec((TCB, NODE_DIM), lambda t: (t + TC_STEPS, 0)),
          pl.BlockSpec((TCB, 1), lambda t: (t, 0)),
          pl.BlockSpec((BATCH // NODE_DIM, NODE_DIM), lambda t: (0, 0)),
          pl.BlockSpec((NUM_LAYER, LAYER_DIM), lambda t: (0, 0)),
          pl.BlockSpec((NODE_DIM, LAYER_DIM), lambda t: (0, 0)),
      ],
      out_specs=pl.BlockSpec((1, 1), lambda t: (0, 0)),
      out_shape=jax.ShapeDtypeStruct((1, 1), jnp.float32),
      scratch_shapes=[pltpu.SMEM((1,), jnp.float32)],
  )(g, g, l.astype(jnp.int32).reshape(BATCH, 1),
    label.reshape(BATCH // NODE_DIM, NODE_DIM), l_emb, w)
  return out[0, 0]


# speedup vs baseline: 1.0285x; 1.0285x over previous
import functools

import jax
import jax.numpy as jnp
from jax import lax
from jax.experimental import pallas as pl
from jax.experimental.pallas import tpu as pltpu
from jax.experimental.pallas import tpu_sc as plsc

NUM_NODES = 100000
NODE_DIM = 128
LAYER_DIM = 64
NUM_LAYER = 8
BATCH = 4096

NC = 2
NS = 16
NW = NC * NS

GB = 2 * BATCH
N_PER_W = GB // NW
CHUNK = 128
N_CHUNKS = N_PER_W // CHUNK


SUB = 64
N_SUB = CHUNK // SUB


def _sc_gather_body(i_hbm, j_hbm, nemb_hbm, out_g_hbm, idx_v, rows_v,
                    isem, gsem, wsem):
  wid = lax.axis_index("s") * NC + lax.axis_index("c")
  base = wid * CHUNK
  st0 = pltpu.async_copy(i_hbm.at[pl.ds(base, CHUNK)], idx_v.at[0], isem)
  st1 = pltpu.async_copy(j_hbm.at[pl.ds(base, CHUNK)], idx_v.at[1], isem)
  st0.wait()
  st1.wait()
  gathers = []
  for h in range(N_CHUNKS):
    for c in range(N_SUB):
      gathers.append(pltpu.async_copy(
          nemb_hbm.at[idx_v.at[h, pl.ds(c * SUB, SUB)]],
          rows_v.at[pl.ds(h * CHUNK + c * SUB, SUB)], gsem))
  writes = []
  for h in range(N_CHUNKS):
    for c in range(N_SUB):
      gathers[h * N_SUB + c].wait()
      writes.append(pltpu.async_copy(
          rows_v.at[pl.ds(h * CHUNK + c * SUB, SUB)],
          out_g_hbm.at[pl.ds(h * BATCH + base + c * SUB, SUB)], wsem))
  for wcp in writes:
    wcp.wait()


@functools.cache
def _sc_gather():
  return pl.kernel(
      _sc_gather_body,
      out_type=jax.ShapeDtypeStruct((GB, NODE_DIM), jnp.float32),
      mesh=plsc.VectorSubcoreMesh(
          core_axis_name="c", subcore_axis_name="s",
          num_cores=NC, num_subcores=NS),
      scratch_types=[
          pltpu.VMEM((N_CHUNKS, CHUNK), jnp.int32),
          pltpu.VMEM((N_PER_W, NODE_DIM), jnp.float32),
          pltpu.SemaphoreType.DMA,
          pltpu.SemaphoreType.DMA,
          pltpu.SemaphoreType.DMA,
      ],
  )


TC_STEPS = 2
TCB = BATCH // TC_STEPS


def _tc_body(gi_ref, gj_ref, l_ref, label_ref, lemb_ref, w_ref, out_ref,
             acc_ref):
  t = pl.program_id(0)
  w = w_ref[...]
  p = jnp.dot(gi_ref[...], w, preferred_element_type=jnp.float32)
  q = jnp.dot(gj_ref[...], w, preferred_element_type=jnp.float32)
  li = l_ref[...]
  oh = (lax.broadcasted_iota(jnp.int32, (TCB, NUM_LAYER), 1)
        == li).astype(jnp.float32)
  lt = jnp.dot(oh, lemb_ref[...], preferred_element_type=jnp.float32)
  part = jnp.sum((lt + p) * (lt + q))

  @pl.when(t == 0)
  def _init():
    acc_ref[0] = part

  @pl.when(jnp.logical_and(t > 0, t < TC_STEPS - 1))
  def _acc():
    acc_ref[0] += part

  @pl.when(t == TC_STEPS - 1)
  def _fini():
    z = label_ref[...] * (acc_ref[0] + part)
    ls = jnp.minimum(z, 0.0) - jnp.log1p(jnp.exp(-jnp.abs(z)))
    out_ref[...] = (-jnp.sum(ls)).reshape(1, 1)


def kernel(i, j, l, label, n_emb, l_emb, w):
  g = _sc_gather()(i.astype(jnp.int32), j.astype(jnp.int32), n_emb)
  out = pl.pallas_call(
      _tc_body,
      grid=(TC_STEPS,),
      in_specs=[
          pl.BlockSpec((TCB, NODE_DIM), lambda t: (t, 0)),
          pl.BlockSpec((TCB, NODE_DIM), lambda t: (t + TC_STEPS, 0)),
          pl.BlockSpec((TCB, 1), lambda t: (t, 0)),
          pl.BlockSpec((BATCH // NODE_DIM, NODE_DIM), lambda t: (0, 0)),
          pl.BlockSpec((NUM_LAYER, LAYER_DIM), lambda t: (0, 0)),
          pl.BlockSpec((NODE_DIM, LAYER_DIM), lambda t: (0, 0)),
      ],
      out_specs=pl.BlockSpec((1, 1), lambda t: (0, 0)),
      out_shape=jax.ShapeDtypeStruct((1, 1), jnp.float32),
      scratch_shapes=[pltpu.SMEM((1,), jnp.float32)],
  )(g, g, l.astype(jnp.int32).reshape(BATCH, 1),
    label.reshape(BATCH // NODE_DIM, NODE_DIM), l_emb, w)
  return out[0, 0]
